# baseline (speedup 1.0000x reference)
import functools

import jax
import jax.numpy as jnp
import numpy as np
from jax import lax
from jax.experimental import pallas as pl
from jax.experimental.pallas import tpu as pltpu
from jax.experimental.pallas import tpu_sc as plsc

_N = 2048
_D = 1024
_H = 2048
_E = 8
_EP = 128
_T = 256
_NT = 16
_P = _NT * _T


def _shift_down_rows(c, s):
    return jnp.concatenate([jnp.zeros((s, _EP), jnp.int32), c[: _N - s, :]], axis=0)


def _shift_right_lanes(v, s):
    return jnp.concatenate([jnp.zeros((1, s), jnp.int32), v[:, : _EP - s]], axis=1)


def _router_body(x_ref, wr_ref, g_ref, pos_ref, cnt_ref):
    z = jnp.dot(x_ref[...], wr_ref[...], preferred_element_type=jnp.float32)
    z = z + g_ref[...]
    m = jnp.max(z, axis=1, keepdims=True)
    p = jnp.exp(z - m)
    y = p / jnp.sum(p, axis=1, keepdims=True)
    my = jnp.max(y, axis=1, keepdims=True)
    lane = lax.broadcasted_iota(jnp.int32, (_N, _EP), 1)
    idx = jnp.min(jnp.where(y == my, lane, _EP), axis=1, keepdims=True)
    onehot = (lane == idx).astype(jnp.int32)
    c = onehot
    s = 1
    while s < _N:
        c = c + _shift_down_rows(c, s)
        s *= 2
    counts = c[_N - 1 : _N, :]
    padded = ((counts + (_T - 1)) // _T) * _T
    o = _shift_right_lanes(padded, 1)
    o = o + _shift_right_lanes(o, 1)
    o = o + _shift_right_lanes(o, 2)
    o = o + _shift_right_lanes(o, 4)
    pos = jnp.sum(jnp.where(onehot == 1, c - 1 + o, 0), axis=1, keepdims=True)
    pos_ref[...] = jnp.broadcast_to(pos, (_N, _EP))
    cnt_ref[...] = jnp.broadcast_to(counts, (8, _EP))


def _router_out_shapes():
    return [
        jax.ShapeDtypeStruct((_N, _EP), jnp.int32),
        jax.ShapeDtypeStruct((8, _EP), jnp.int32),
    ]


_DEPTH = 3
_NCH = 4


def _w1_pair(w1_hbm, w1b, e, s, c):
    blk = _D // _NCH
    return (w1_hbm.at[e, pl.ds(c * blk, blk), :], w1b.at[s, pl.ds(c * blk, blk), :])


def _w2_pair(w2_hbm, w2b, e, s, c):
    blk = _H // _NCH
    return (w2_hbm.at[e, pl.ds(c * blk, blk), :], w2b.at[s, pl.ds(c * blk, blk), :])


def _ffn_body(meta_ref, xs_ref, w1_hbm, w2_hbm, b1_ref, b2_ref, out_ref,
              w1b, w2b, sems):
    w = pl.program_id(0)
    e = meta_ref[w]
    act = meta_ref[_NT + w]
    ordw = meta_ref[2 * _NT + w]
    first = meta_ref[3 * _NT + w]
    nxte = meta_ref[4 * _NT + w]
    hasnx = meta_ref[5 * _NT + w]
    nxte2 = meta_ref[6 * _NT + w]
    hasnx2 = meta_ref[7 * _NT + w]
    slot = jax.lax.rem(ordw, _DEPTH)

    def issue(ee, ss):
        for c in range(_NCH):
            src, dst = _w1_pair(w1_hbm, w1b, ee, ss, c)
            pltpu.make_async_copy(src, dst, sems.at[ss]).start()
            src, dst = _w2_pair(w2_hbm, w2b, ee, ss, c)
            pltpu.make_async_copy(src, dst, sems.at[ss]).start()

    def drain(ee, ss):
        for c in range(_NCH):
            src, dst = _w1_pair(w1_hbm, w1b, ee, ss, c)
            pltpu.make_async_copy(src, dst, sems.at[ss]).wait()
            src, dst = _w2_pair(w2_hbm, w2b, ee, ss, c)
            pltpu.make_async_copy(src, dst, sems.at[ss]).wait()

    @pl.when((w == 0) & (act == 1))
    def _():
        issue(e, slot)

    @pl.when((w == 0) & (hasnx == 1))
    def _():
        issue(nxte, jax.lax.rem(ordw + 1, _DEPTH))

    @pl.when((first == 1) & (hasnx2 == 1))
    def _():
        issue(nxte2, jax.lax.rem(ordw + 2, _DEPTH))

    @pl.when((first == 1) & (act == 1))
    def _():
        drain(e, slot)

    @pl.when(act == 1)
    def _():
        x = xs_ref[...]
        h = jnp.dot(x, w1b[slot], preferred_element_type=jnp.float32)
        h = jnp.maximum(h + b1_ref[0, 0, :][None, :], 0.0)
        y = jnp.dot(h, w2b[slot], preferred_element_type=jnp.float32)
        out_ref[...] = y + b2_ref[0, 0, :][None, :]


def _ffn_grid_spec():
    return pltpu.PrefetchScalarGridSpec(
        num_scalar_prefetch=1,
        grid=(_NT,),
        in_specs=[
            pl.BlockSpec((_T, _D), lambda w, m: (w, 0)),
            pl.BlockSpec(memory_space=pltpu.MemorySpace.HBM),
            pl.BlockSpec(memory_space=pltpu.MemorySpace.HBM),
            pl.BlockSpec((1, 1, _H), lambda w, m: (m[w], 0, 0)),
            pl.BlockSpec((1, 1, _D), lambda w, m: (m[w], 0, 0)),
        ],
        out_specs=pl.BlockSpec((_T, _D), lambda w, m: (w, 0)),
        scratch_shapes=[
            pltpu.VMEM((_DEPTH, _D, _H), jnp.float32),
            pltpu.VMEM((_DEPTH, _H, _D), jnp.float32),
            pltpu.SemaphoreType.DMA((_DEPTH,)),
        ],
    )


def _sc_info():
    info = plsc.get_sparse_core_info()
    return info.num_cores, info.num_subcores


def _scatter_rows(x, pos):
    nc, ns = _sc_info()
    bpw = _N // (nc * ns)
    mesh = plsc.VectorSubcoreMesh(core_axis_name="c", subcore_axis_name="s")

    @functools.partial(
        pl.kernel,
        mesh=mesh,
        out_type=jax.ShapeDtypeStruct((_P, _D), jnp.float32),
        scratch_types=[
            pltpu.VMEM((bpw,), jnp.int32),
            pltpu.VMEM((bpw, _D), jnp.float32),
            pltpu.SemaphoreType.DMA,
        ],
    )
    def k(x_hbm, pos_hbm, out_hbm, idx_v, rows_v, sem):
        wid = lax.axis_index("s") * nc + lax.axis_index("c")
        base = wid * bpw
        pltpu.sync_copy(pos_hbm.at[pl.ds(base, bpw)], idx_v)
        pltpu.sync_copy(x_hbm.at[pl.ds(base, bpw)], rows_v)
        pltpu.async_copy(rows_v, out_hbm.at[idx_v], sem).wait()

    return k(x, pos)


def _gather_rows(ys, pos):
    nc, ns = _sc_info()
    bpw = _N // (nc * ns)
    mesh = plsc.VectorSubcoreMesh(core_axis_name="c", subcore_axis_name="s")

    @functools.partial(
        pl.kernel,
        mesh=mesh,
        out_type=jax.ShapeDtypeStruct((_N, _D), jnp.float32),
        scratch_types=[
            pltpu.VMEM((bpw,), jnp.int32),
            pltpu.VMEM((bpw, _D), jnp.float32),
            pltpu.SemaphoreType.DMA,
        ],
    )
    def k(ys_hbm, pos_hbm, out_hbm, idx_v, rows_v, sem):
        wid = lax.axis_index("s") * nc + lax.axis_index("c")
        base = wid * bpw
        pltpu.sync_copy(pos_hbm.at[pl.ds(base, bpw)], idx_v)
        pltpu.async_copy(ys_hbm.at[idx_v], rows_v, sem).wait()
        pltpu.sync_copy(rows_v, out_hbm.at[pl.ds(base, bpw)])

    return k(ys, pos)


def _ffn_meta(counts):
    pc = ((counts + (_T - 1)) // _T) * _T
    pend = jnp.cumsum(pc)
    tstart = jnp.arange(_NT, dtype=jnp.int32) * _T
    eot = jnp.minimum(
        jnp.sum((tstart[:, None] >= pend[None, :]).astype(jnp.int32), axis=1),
        _E - 1).astype(jnp.int32)
    act = (tstart < pend[_E - 1]).astype(jnp.int32)
    nz = (counts > 0).astype(jnp.int32)
    n_uniq = jnp.sum(nz)
    ordx = jnp.cumsum(nz) - nz
    uniq = jnp.argsort(jnp.where(nz == 1, jnp.arange(_E), _E)).astype(jnp.int32)
    ordw = ordx[eot].astype(jnp.int32)
    prev_eot = jnp.concatenate([jnp.full((1,), -1, jnp.int32), eot[:-1]])
    first = ((eot != prev_eot) & (act == 1)).astype(jnp.int32)
    nxte = uniq[jnp.minimum(ordw + 1, _E - 1)]
    hasnx = ((ordw + 1 < n_uniq) & (act == 1)).astype(jnp.int32)
    nxte2 = uniq[jnp.minimum(ordw + 2, _E - 1)]
    hasnx2 = ((ordw + 2 < n_uniq) & (act == 1)).astype(jnp.int32)
    return jnp.concatenate([eot, act, ordw, first, nxte, hasnx, nxte2, hasnx2])


_GUMBEL_CACHE = []


def _np_threefry2x32(k0, k1, x0, x1):
    r0 = (13, 15, 26, 6)
    r1 = (17, 29, 16, 24)
    ks2 = np.uint32(0x1BD11BDA) ^ k0 ^ k1
    x0 = (x0 + k0).astype(np.uint32)
    x1 = (x1 + k1).astype(np.uint32)
    inject = ((k1, ks2), (ks2, k0), (k0, k1), (k1, ks2), (ks2, k0))
    rots = (r0, r1, r0, r1, r0)
    for i in range(5):
        for r in rots[i]:
            x0 = (x0 + x1).astype(np.uint32)
            x1 = ((x1 << np.uint32(r)) | (x1 >> np.uint32(32 - r))).astype(np.uint32) ^ x0
        x0 = (x0 + inject[i][0]).astype(np.uint32)
        x1 = (x1 + inject[i][1] + np.uint32(i + 1)).astype(np.uint32)
    return x0, x1


def _gumbel_const():
    if not _GUMBEL_CACHE:
        idx = np.arange(_N * _E, dtype=np.uint32)
        y0, y1 = _np_threefry2x32(np.uint32(0), np.uint32(42),
                                  np.zeros_like(idx), idx)
        bits = y0 ^ y1
        fl = ((bits >> np.uint32(9)) | np.uint32(0x3F800000)).view(np.float32)
        fl = fl - np.float32(1.0)
        mn, mx = np.float32(1e-10), np.float32(1.0)
        u = np.maximum(mn, fl * (mx - mn) + mn).reshape(_N, _E)
        gum = -np.log(-np.log(u))
        pad = np.full((_N, _EP), -1e30, np.float32)
        pad[:, :_E] = gum
        _GUMBEL_CACHE.append(pad)
    return _GUMBEL_CACHE[0]


def kernel(x, Wr, br, W1, b1, W2, b2):
    g_pad = jnp.asarray(_gumbel_const()) + jnp.pad(br, (0, _EP - _E))[None, :]
    wr_pad = jnp.zeros((_D, _EP), jnp.float32).at[:, :_E].set(Wr)

    pos_b, cnt_b = pl.pallas_call(
        _router_body,
        out_shape=_router_out_shapes(),
    )(x, wr_pad, g_pad)
    pos = pos_b[:, 0]
    counts = cnt_b[0, :_E]

    xs = _scatter_rows(x, pos)

    meta = _ffn_meta(counts)
    ys = pl.pallas_call(
        _ffn_body,
        grid_spec=_ffn_grid_spec(),
        out_shape=jax.ShapeDtypeStruct((_P, _D), jnp.float32),
    )(meta, xs, W1, W2, b1.reshape(_E, 1, _H), b2.reshape(_E, 1, _D))

    return _gather_rows(ys, pos)
